# baseline (speedup 1.0000x reference)
import jax
import jax.numpy as jnp
from jax import lax
from jax.experimental import pallas as pl
from jax.experimental.pallas import tpu as pltpu
from jax.experimental.pallas import tpu_sc as plsc

N_NODES = 10000
N_EDGES = 320000
F_IN = 128
F_E = 16
H = 256
HH = H // 2
DEPTH = 3

NC = 2
NS = 16
LANES = 16

CHUNK = 80
EDGES_PER_TILE = N_EDGES // NS
NCHUNK = EDGES_PER_TILE // CHUNK

SEG_START = (0, 3336, 6672)
SEG_SIZE = (3336, 3336, 3328)
NSEG = 3
DUMP = 3336
ACC_ROWS = 3344
FL_ROWS = 208
ZROWS = 8

BLK = 2000
NBLK = EDGES_PER_TILE // BLK
NCH_P = 256
SLOTS = NCH_P * CHUNK


MBLK = 2000
EBLK = 4000


def _proj_body(x_ref, w_ref, b_ref, u1_ref, b1_ref, h_ref, ha_ref, hb_ref):
    h = jnp.maximum(
        jnp.dot(x_ref[...], w_ref[...], preferred_element_type=jnp.float32)
        + b_ref[...], 0.0)
    h_ref[...] = h
    hu1 = jnp.dot(h, u1_ref[...], preferred_element_type=jnp.float32) + b1_ref[...]
    ha_ref[...] = hu1[:, :HH]
    hb_ref[...] = hu1[:, HH:]


def _input_projection(x, W, b, U1, b1):
    grid = (N_NODES // MBLK,)
    return pl.pallas_call(
        _proj_body,
        grid=grid,
        in_specs=[
            pl.BlockSpec((MBLK, F_IN), lambda i: (i, 0)),
            pl.BlockSpec((F_IN, H), lambda i: (0, 0)),
            pl.BlockSpec((1, H), lambda i: (0, 0)),
            pl.BlockSpec((H, H), lambda i: (0, 0)),
            pl.BlockSpec((1, H), lambda i: (0, 0)),
        ],
        out_specs=[
            pl.BlockSpec((MBLK, H), lambda i: (i, 0)),
            pl.BlockSpec((MBLK, HH), lambda i: (i, 0)),
            pl.BlockSpec((MBLK, HH), lambda i: (i, 0)),
        ],
        out_shape=[
            jax.ShapeDtypeStruct((N_NODES, H), jnp.float32),
            jax.ShapeDtypeStruct((N_NODES, HH), jnp.float32),
            jax.ShapeDtypeStruct((N_NODES, HH), jnp.float32),
        ],
    )(x, W, b.reshape(1, H), U1, b1.reshape(1, H))


def _edge_body(ea_ref, u2_ref, b2_ref, ea_out, eb_out):
    e2 = jnp.dot(ea_ref[...], u2_ref[...], preferred_element_type=jnp.float32) \
        + b2_ref[...]
    ea_out[...] = e2[:, :HH]
    eb_out[...] = e2[:, HH:]


def _edge_projection(edge_attr, U2, b2):
    grid = (N_EDGES // EBLK,)
    return pl.pallas_call(
        _edge_body,
        grid=grid,
        in_specs=[
            pl.BlockSpec((EBLK, F_E), lambda i: (i, 0)),
            pl.BlockSpec((F_E, H), lambda i: (0, 0)),
            pl.BlockSpec((1, H), lambda i: (0, 0)),
        ],
        out_specs=[
            pl.BlockSpec((EBLK, HH), lambda i: (i, 0)),
            pl.BlockSpec((EBLK, HH), lambda i: (i, 0)),
        ],
        out_shape=[
            jax.ShapeDtypeStruct((N_EDGES, HH), jnp.float32),
            jax.ShapeDtypeStruct((N_EDGES, HH), jnp.float32),
        ],
    )(edge_attr, U2, b2.reshape(1, H))


def _update_body(h_ref, aa_ref, ab_ref, u0_ref, b0_ref, u1_ref, b1_ref,
                 hn_ref, ha_ref, hb_ref):
    agg = jnp.concatenate([aa_ref[...], ab_ref[...]], axis=1)
    hn = jnp.maximum(
        jnp.dot(h_ref[...], u0_ref[...], preferred_element_type=jnp.float32)
        + b0_ref[...] + agg, 0.0)
    hn_ref[...] = hn
    hu1 = jnp.dot(hn, u1_ref[...], preferred_element_type=jnp.float32) + b1_ref[...]
    ha_ref[...] = hu1[:, :HH]
    hb_ref[...] = hu1[:, HH:]


def _node_update(h, agga, aggb, U0, b0, U1, b1):
    grid = (N_NODES // MBLK,)
    return pl.pallas_call(
        _update_body,
        grid=grid,
        in_specs=[
            pl.BlockSpec((MBLK, H), lambda i: (i, 0)),
            pl.BlockSpec((MBLK, HH), lambda i: (i, 0)),
            pl.BlockSpec((MBLK, HH), lambda i: (i, 0)),
            pl.BlockSpec((H, H), lambda i: (0, 0)),
            pl.BlockSpec((1, H), lambda i: (0, 0)),
            pl.BlockSpec((H, H), lambda i: (0, 0)),
            pl.BlockSpec((1, H), lambda i: (0, 0)),
        ],
        out_specs=[
            pl.BlockSpec((MBLK, H), lambda i: (i, 0)),
            pl.BlockSpec((MBLK, HH), lambda i: (i, 0)),
            pl.BlockSpec((MBLK, HH), lambda i: (i, 0)),
        ],
        out_shape=[
            jax.ShapeDtypeStruct((N_NODES, H), jnp.float32),
            jax.ShapeDtypeStruct((N_NODES, HH), jnp.float32),
            jax.ShapeDtypeStruct((N_NODES, HH), jnp.float32),
        ],
    )(h, agga, aggb, U0, b0.reshape(1, H), U1, b1.reshape(1, H))


def _edge_pass_body(hu1a, hu1b, e2a, e2b, psrc, pdstl, pidx, counts_in,
                    agga, aggb,
                    idx_sv, idx_dv, idx_ev, rows, e2v, zbuf, acc, cbuf,
                    sem_in0, sem_in1, sem_sc0, sem_sc1):
    c = lax.axis_index("c")
    s = lax.axis_index("s")
    sem_in = (sem_in0, sem_in1)
    sem_sc = (sem_sc0, sem_sc1)

    pltpu.sync_copy(counts_in.at[s], cbuf)
    cv = cbuf[0, pl.ds(0, LANES)]
    nc0 = cv[0]
    nc01 = cv[1]
    nct = cv[2]

    for k in range(HH // LANES):
        for r in range(ZROWS):
            zbuf[r, pl.ds(k * LANES, LANES)] = jnp.zeros((LANES,), jnp.float32)

    def zero_acc():
        for blk in range(FL_ROWS // ZROWS):
            pltpu.sync_copy(zbuf,
                            acc.at[pl.ds(s * FL_ROWS + blk * ZROWS, ZROWS)])

        @pl.when(s == 0)
        def _():
            for blk in range((ACC_ROWS - NS * FL_ROWS) // ZROWS):
                pltpu.sync_copy(
                    zbuf,
                    acc.at[pl.ds(NS * FL_ROWS + blk * ZROWS, ZROWS)])

    zero_acc()
    plsc.subcore_barrier()

    def load_idx(g, b):
        pltpu.sync_copy(psrc.at[s].at[g], idx_sv.at[b])
        pltpu.sync_copy(pdstl.at[s].at[g], idx_dv.at[b])
        pltpu.sync_copy(pidx.at[s].at[g], idx_ev.at[b])

    def issue_in(b):
        @pl.when(c == 0)
        def _():
            pltpu.async_copy(hu1a.at[idx_sv.at[b]], rows.at[b], sem_in[b])
            pltpu.async_copy(e2a.at[idx_ev.at[b]], e2v.at[b], sem_in[b])

        @pl.when(c == 1)
        def _():
            pltpu.async_copy(hu1b.at[idx_sv.at[b]], rows.at[b], sem_in[b])
            pltpu.async_copy(e2b.at[idx_ev.at[b]], e2v.at[b], sem_in[b])

    def wait_in(b):
        pltpu.make_async_copy(hu1a.at[idx_sv.at[b]], rows.at[b],
                              sem_in[b]).wait()
        pltpu.make_async_copy(e2a.at[idx_ev.at[b]], e2v.at[b],
                              sem_in[b]).wait()

    def wait_sc(b):
        pltpu.make_async_copy(rows.at[b], acc.at[idx_dv.at[b]],
                              sem_sc[b]).wait()

    def run_pass(first, n_p):
        @pl.when(n_p > 0)
        def _():
            load_idx(first, 0)
            issue_in(0)

        def step(g, b):
            @pl.when(g + 1 < n_p)
            def _():
                @pl.when(g >= 1)
                def _():
                    wait_sc(1 - b)
                load_idx(first + g + 1, 1 - b)
                issue_in(1 - b)

            wait_in(b)

            @plsc.parallel_loop(0, CHUNK, 1, unroll=2)
            def vb(r):
                for k in range(HH // LANES):
                    off = k * LANES
                    v = rows[b, r, pl.ds(off, LANES)] \
                        + e2v[b, r, pl.ds(off, LANES)]
                    rows[b, r, pl.ds(off, LANES)] = jnp.maximum(v, 0.0)

            pltpu.async_copy(rows.at[b], acc.at[idx_dv.at[b]], sem_sc[b],
                             add=True)

        def pair(t, _):
            step(2 * t, 0)
            step(2 * t + 1, 1)
            return 0

        lax.fori_loop(0, n_p // 2, pair, 0)

        @pl.when(n_p > 0)
        def _():
            wait_sc(0)
            wait_sc(1)

    def flush(k):
        rem = SEG_SIZE[k] - NS * FL_ROWS

        def copy_out(out):
            pltpu.sync_copy(acc.at[pl.ds(s * FL_ROWS, FL_ROWS)],
                            out.at[pl.ds(SEG_START[k] + s * FL_ROWS,
                                         FL_ROWS)])
            if rem:
                @pl.when(s == 0)
                def _():
                    pltpu.sync_copy(
                        acc.at[pl.ds(NS * FL_ROWS, rem)],
                        out.at[pl.ds(SEG_START[k] + NS * FL_ROWS, rem)])

        @pl.when(c == 0)
        def _():
            copy_out(agga)

        @pl.when(c == 1)
        def _():
            copy_out(aggb)

    run_pass(jnp.int32(0), nc0)
    plsc.subcore_barrier()
    flush(0)
    zero_acc()
    plsc.subcore_barrier()
    run_pass(nc0, nc01 - nc0)
    plsc.subcore_barrier()
    flush(1)
    zero_acc()
    plsc.subcore_barrier()
    run_pass(nc01, nct - nc01)
    plsc.subcore_barrier()
    flush(2)


_EDGE_PASS_CACHE = {}


def _edge_pass_kernel():
    if "k" not in _EDGE_PASS_CACHE:
        _EDGE_PASS_CACHE["k"] = pl.kernel(
            _edge_pass_body,
            out_type=[
                jax.ShapeDtypeStruct((N_NODES, HH), jnp.float32),
                jax.ShapeDtypeStruct((N_NODES, HH), jnp.float32),
            ],
            mesh=plsc.VectorSubcoreMesh(core_axis_name="c",
                                        subcore_axis_name="s",
                                        num_cores=NC, num_subcores=NS),
            scratch_types=[
                pltpu.VMEM((2, CHUNK), jnp.int32),
                pltpu.VMEM((2, CHUNK), jnp.int32),
                pltpu.VMEM((2, CHUNK), jnp.int32),
                pltpu.VMEM((2, CHUNK, HH), jnp.float32),
                pltpu.VMEM((2, CHUNK, HH), jnp.float32),
                pltpu.VMEM((ZROWS, HH), jnp.float32),
                pltpu.VMEM_SHARED((ACC_ROWS, HH), jnp.float32),
                pltpu.VMEM((1, LANES), jnp.int32),
                pltpu.SemaphoreType.DMA,
                pltpu.SemaphoreType.DMA,
                pltpu.SemaphoreType.DMA,
                pltpu.SemaphoreType.DMA,
            ],
        )
    return _EDGE_PASS_CACHE["k"]


def _prepare_edges(src, dst):
    dst2 = dst.reshape(NS, EDGES_PER_TILE)
    m1 = (dst2 >= SEG_START[1]).astype(jnp.int32)
    m2 = (dst2 >= SEG_START[2]).astype(jnp.int32)
    m0i = 1 - m1
    m1i = m1 - m2

    def cumsum2(m):
        mc = m.reshape(NS, EDGES_PER_TILE // CHUNK, CHUNK)
        within = jnp.cumsum(mc, axis=2)
        csum = within[:, :, -1]
        base = jnp.cumsum(csum, axis=1) - csum
        return (base[:, :, None] + within).reshape(NS, EDGES_PER_TILE)

    c0 = cumsum2(m0i)
    c1 = cumsum2(m1i)
    c2 = cumsum2(m2)
    cnt0, cnt1, cnt2 = c0[:, -1], c1[:, -1], c2[:, -1]

    def even_chunks(cnt):
        ch = (cnt + CHUNK - 1) // CHUNK
        return ch + (ch & 1)

    nc0 = even_chunks(cnt0)
    nc1 = even_chunks(cnt1)
    nc2 = even_chunks(cnt2)
    b1 = nc0 * CHUNK
    b2 = (nc0 + nc1) * CHUNK
    seg = m1 + m2
    pos = jnp.where(seg == 0, c0 - m0i,
                    jnp.where(seg == 1, b1[:, None] + c1 - m1i,
                              b2[:, None] + c2 - m2))
    gpos = (jnp.arange(NS, dtype=jnp.int32)[:, None] * SLOTS + pos).reshape(-1)
    eid = jnp.arange(NS * EDGES_PER_TILE, dtype=jnp.int32)
    pidx = jnp.zeros((NS * SLOTS,), jnp.int32).at[gpos].set(
        eid, unique_indices=True, mode="promise_in_bounds").reshape(NS, SLOTS)
    slotch = jnp.arange(SLOTS, dtype=jnp.int32) // CHUNK
    segsl = ((slotch[None, :] >= nc0[:, None]).astype(jnp.int32)
             + (slotch[None, :] >= (nc0 + nc1)[:, None]).astype(jnp.int32))
    seg_base = jnp.where(segsl == 0, 0,
                         jnp.where(segsl == 1, b1[:, None], b2[:, None]))
    cnt_sel = jnp.where(segsl == 0, cnt0[:, None],
                        jnp.where(segsl == 1, cnt1[:, None], cnt2[:, None]))
    valid = (jnp.arange(SLOTS, dtype=jnp.int32)[None, :] - seg_base) < cnt_sel
    flat_pidx = pidx.reshape(-1)
    psrc = src.reshape(-1).at[flat_pidx].get(
        mode="promise_in_bounds").reshape(NS, SLOTS)
    pdst = dst.reshape(-1).at[flat_pidx].get(
        mode="promise_in_bounds").reshape(NS, SLOTS)
    pdstl = jnp.where(valid, pdst - SEG_START[1] * segsl, DUMP)
    nc01 = nc0 + nc1
    nct = nc01 + nc2
    lane = jnp.arange(LANES, dtype=jnp.int32)[None, :]
    counts = jnp.where(lane < 1, nc0[:, None],
                       jnp.where(lane < 2, nc01[:, None], nct[:, None]))
    return (psrc.reshape(NS, NCH_P, CHUNK),
            pdstl.astype(jnp.int32).reshape(NS, NCH_P, CHUNK),
            pidx.reshape(NS, NCH_P, CHUNK),
            counts.astype(jnp.int32).reshape(NS, 1, LANES))


def kernel(x, edge_index, edge_attr, W, b, U0, b0, U1, b1, U2, b2):
    psrc, pdstl, pidx, counts = _prepare_edges(edge_index[0], edge_index[1])

    h, ha, hb = _input_projection(x, W, b, U1, b1)
    e2a, e2b = _edge_projection(edge_attr, U2, b2)

    def depth_body(d, carry):
        h, ha, hb = carry
        agga, aggb = _edge_pass_kernel()(
            ha, hb, e2a, e2b, psrc, pdstl, pidx, counts)
        return tuple(_node_update(h, agga, aggb, U0, b0, U1, b1))

    h, _, _ = lax.fori_loop(0, DEPTH, depth_body, (h, ha, hb))
    return h


# speedup vs baseline: 1.4890x; 1.4890x over previous
import jax
import jax.numpy as jnp
from jax import lax
from jax.experimental import pallas as pl
from jax.experimental.pallas import tpu as pltpu
from jax.experimental.pallas import tpu_sc as plsc

N_NODES = 10000
N_EDGES = 320000
F_IN = 128
F_E = 16
H = 256
HH = H // 2
DEPTH = 3

NC = 2
NS = 16
LANES = 16

CHUNK = 80
EDGES_PER_TILE = N_EDGES // NS
NCHUNK = EDGES_PER_TILE // CHUNK

NH = N_NODES // 2
ACC_ROWS = NH + 8
FL_ROWS = 312
FL_REM = NH - FL_ROWS * NS
ZROWS = 8

MBLK = 2000
EBLK = 4000


def _proj_body(x_ref, w_ref, b_ref, u1_ref, b1_ref, h_ref, ha_ref, hb_ref):
    h = jnp.maximum(
        jnp.dot(x_ref[...], w_ref[...], preferred_element_type=jnp.float32)
        + b_ref[...], 0.0)
    h_ref[...] = h
    hu1 = jnp.dot(h, u1_ref[...], preferred_element_type=jnp.float32) + b1_ref[...]
    ha_ref[...] = hu1[:, :HH]
    hb_ref[...] = hu1[:, HH:]


def _input_projection(x, W, b, U1, b1):
    grid = (N_NODES // MBLK,)
    return pl.pallas_call(
        _proj_body,
        grid=grid,
        in_specs=[
            pl.BlockSpec((MBLK, F_IN), lambda i: (i, 0)),
            pl.BlockSpec((F_IN, H), lambda i: (0, 0)),
            pl.BlockSpec((1, H), lambda i: (0, 0)),
            pl.BlockSpec((H, H), lambda i: (0, 0)),
            pl.BlockSpec((1, H), lambda i: (0, 0)),
        ],
        out_specs=[
            pl.BlockSpec((MBLK, H), lambda i: (i, 0)),
            pl.BlockSpec((MBLK, HH), lambda i: (i, 0)),
            pl.BlockSpec((MBLK, HH), lambda i: (i, 0)),
        ],
        out_shape=[
            jax.ShapeDtypeStruct((N_NODES, H), jnp.float32),
            jax.ShapeDtypeStruct((N_NODES, HH), jnp.float32),
            jax.ShapeDtypeStruct((N_NODES, HH), jnp.float32),
        ],
    )(x, W, b.reshape(1, H), U1, b1.reshape(1, H))


def _edge_body(ea_ref, u2_ref, b2_ref, ea_out, eb_out):
    e2 = jnp.dot(ea_ref[...], u2_ref[...], preferred_element_type=jnp.float32) \
        + b2_ref[...]
    ea_out[...] = e2[:, :HH]
    eb_out[...] = e2[:, HH:]


def _edge_projection(edge_attr, U2, b2):
    grid = (N_EDGES // EBLK,)
    return pl.pallas_call(
        _edge_body,
        grid=grid,
        in_specs=[
            pl.BlockSpec((EBLK, F_E), lambda i: (i, 0)),
            pl.BlockSpec((F_E, H), lambda i: (0, 0)),
            pl.BlockSpec((1, H), lambda i: (0, 0)),
        ],
        out_specs=[
            pl.BlockSpec((EBLK, HH), lambda i: (i, 0)),
            pl.BlockSpec((EBLK, HH), lambda i: (i, 0)),
        ],
        out_shape=[
            jax.ShapeDtypeStruct((N_EDGES, HH), jnp.float32),
            jax.ShapeDtypeStruct((N_EDGES, HH), jnp.float32),
        ],
    )(edge_attr, U2, b2.reshape(1, H))


def _update_body(h_ref, aa_ref, ab_ref, u0_ref, b0_ref, u1_ref, b1_ref,
                 hn_ref, ha_ref, hb_ref):
    agg = jnp.concatenate([aa_ref[...], ab_ref[...]], axis=1)
    hn = jnp.maximum(
        jnp.dot(h_ref[...], u0_ref[...], preferred_element_type=jnp.float32)
        + b0_ref[...] + agg, 0.0)
    hn_ref[...] = hn
    hu1 = jnp.dot(hn, u1_ref[...], preferred_element_type=jnp.float32) + b1_ref[...]
    ha_ref[...] = hu1[:, :HH]
    hb_ref[...] = hu1[:, HH:]


def _node_update(h, agga, aggb, U0, b0, U1, b1):
    grid = (N_NODES // MBLK,)
    return pl.pallas_call(
        _update_body,
        grid=grid,
        in_specs=[
            pl.BlockSpec((MBLK, H), lambda i: (i, 0)),
            pl.BlockSpec((MBLK, HH), lambda i: (i, 0)),
            pl.BlockSpec((MBLK, HH), lambda i: (i, 0)),
            pl.BlockSpec((H, H), lambda i: (0, 0)),
            pl.BlockSpec((1, H), lambda i: (0, 0)),
            pl.BlockSpec((H, H), lambda i: (0, 0)),
            pl.BlockSpec((1, H), lambda i: (0, 0)),
        ],
        out_specs=[
            pl.BlockSpec((MBLK, H), lambda i: (i, 0)),
            pl.BlockSpec((MBLK, HH), lambda i: (i, 0)),
            pl.BlockSpec((MBLK, HH), lambda i: (i, 0)),
        ],
        out_shape=[
            jax.ShapeDtypeStruct((N_NODES, H), jnp.float32),
            jax.ShapeDtypeStruct((N_NODES, HH), jnp.float32),
            jax.ShapeDtypeStruct((N_NODES, HH), jnp.float32),
        ],
    )(h, agga, aggb, U0, b0.reshape(1, H), U1, b1.reshape(1, H))


def _edge_pass_body(hu1a, hu1b, e2a, e2b, src3d, dst3d,
                    agga, aggb,
                    idx_sv, idx_dv, idx_buf, rows, e2v, zbuf, acc,
                    sem_in0, sem_in1, sem_sc0, sem_sc1, sem_ix0, sem_ix1):
    c = lax.axis_index("c")
    s = lax.axis_index("s")
    sem_in = (sem_in0, sem_in1)
    sem_sc = (sem_sc0, sem_sc1)
    sem_ix = (sem_ix0, sem_ix1)

    for k in range(HH // LANES):
        for r in range(ZROWS):
            zbuf[r, pl.ds(k * LANES, LANES)] = jnp.zeros((LANES,), jnp.float32)

    def zero_acc():
        for blk in range(FL_ROWS // ZROWS):
            pltpu.sync_copy(zbuf,
                            acc.at[pl.ds(s * FL_ROWS + blk * ZROWS, ZROWS)])

        @pl.when(s == 0)
        def _():
            for blk in range((ACC_ROWS - NS * FL_ROWS) // ZROWS):
                pltpu.sync_copy(
                    zbuf,
                    acc.at[pl.ds(NS * FL_ROWS + blk * ZROWS, ZROWS)])

    zero_acc()
    plsc.subcore_barrier()

    def load_idx(g, b):
        pltpu.async_copy(src3d.at[s].at[g], idx_sv.at[b], sem_ix[b])
        pltpu.async_copy(dst3d.at[s].at[g], idx_dv.at[b], sem_ix[b])

    def wait_idx(b):
        pltpu.make_async_copy(src3d.at[s].at[0], idx_sv.at[b],
                              sem_ix[b]).wait()
        pltpu.make_async_copy(dst3d.at[s].at[0], idx_dv.at[b],
                              sem_ix[b]).wait()

    def issue_in(g, b):
        base = s * EDGES_PER_TILE + g * CHUNK

        @pl.when(c == 0)
        def _():
            pltpu.async_copy(hu1a.at[idx_sv.at[b]], rows.at[b], sem_in[b])
            pltpu.async_copy(e2a.at[pl.ds(base, CHUNK)], e2v.at[b], sem_in[b])

        @pl.when(c == 1)
        def _():
            pltpu.async_copy(hu1b.at[idx_sv.at[b]], rows.at[b], sem_in[b])
            pltpu.async_copy(e2b.at[pl.ds(base, CHUNK)], e2v.at[b], sem_in[b])

    def wait_in(b):
        pltpu.make_async_copy(hu1a.at[idx_sv.at[b]], rows.at[b],
                              sem_in[b]).wait()
        pltpu.make_async_copy(e2a.at[pl.ds(0, CHUNK)], e2v.at[b],
                              sem_in[b]).wait()

    def wait_sc(b):
        pltpu.make_async_copy(rows.at[b], acc.at[idx_buf.at[b]],
                              sem_sc[b]).wait()

    for p in range(2):
        load_idx(0, 0)
        wait_idx(0)
        issue_in(0, 0)
        load_idx(1, 1)

        def step(g, b):
            @pl.when(g + 1 < NCHUNK)
            def _():
                @pl.when(g >= 1)
                def _():
                    wait_sc(1 - b)
                wait_idx(1 - b)
                issue_in(g + 1, 1 - b)

            wait_in(b)

            for k in range(CHUNK // LANES):
                off = k * LANES
                v = idx_dv[b, pl.ds(off, LANES)] - (p * NH)
                ok = (v >= 0) & (v < NH)
                idx_buf[b, pl.ds(off, LANES)] = jnp.where(ok, v, NH)

            @pl.when(g + 2 < NCHUNK)
            def _():
                load_idx(g + 2, b)

            @plsc.parallel_loop(0, CHUNK, 1, unroll=2)
            def vb(r):
                for k in range(HH // LANES):
                    off = k * LANES
                    v = rows[b, r, pl.ds(off, LANES)] \
                        + e2v[b, r, pl.ds(off, LANES)]
                    rows[b, r, pl.ds(off, LANES)] = jnp.maximum(v, 0.0)

            pltpu.async_copy(rows.at[b], acc.at[idx_buf.at[b]], sem_sc[b],
                             add=True)

        def pair(t, _):
            step(2 * t, 0)
            step(2 * t + 1, 1)
            return 0

        lax.fori_loop(0, NCHUNK // 2, pair, 0)
        wait_sc(0)
        wait_sc(1)
        plsc.subcore_barrier()

        def flush(out):
            pltpu.sync_copy(acc.at[pl.ds(s * FL_ROWS, FL_ROWS)],
                            out.at[pl.ds(p * NH + s * FL_ROWS, FL_ROWS)])

            @pl.when(s == 0)
            def _():
                pltpu.sync_copy(acc.at[pl.ds(NS * FL_ROWS, FL_REM)],
                                out.at[pl.ds(p * NH + NS * FL_ROWS, FL_REM)])

        @pl.when(c == 0)
        def _():
            flush(agga)

        @pl.when(c == 1)
        def _():
            flush(aggb)

        if p == 0:
            zero_acc()
            plsc.subcore_barrier()


_EDGE_PASS_CACHE = {}


def _edge_pass_kernel():
    if "k" not in _EDGE_PASS_CACHE:
        _EDGE_PASS_CACHE["k"] = pl.kernel(
            _edge_pass_body,
            out_type=[
                jax.ShapeDtypeStruct((N_NODES, HH), jnp.float32),
                jax.ShapeDtypeStruct((N_NODES, HH), jnp.float32),
            ],
            mesh=plsc.VectorSubcoreMesh(core_axis_name="c",
                                        subcore_axis_name="s",
                                        num_cores=NC, num_subcores=NS),
            scratch_types=[
                pltpu.VMEM((2, CHUNK), jnp.int32),
                pltpu.VMEM((2, CHUNK), jnp.int32),
                pltpu.VMEM((2, CHUNK), jnp.int32),
                pltpu.VMEM((2, CHUNK, HH), jnp.float32),
                pltpu.VMEM((2, CHUNK, HH), jnp.float32),
                pltpu.VMEM((ZROWS, HH), jnp.float32),
                pltpu.VMEM_SHARED((ACC_ROWS, HH), jnp.float32),
                pltpu.SemaphoreType.DMA,
                pltpu.SemaphoreType.DMA,
                pltpu.SemaphoreType.DMA,
                pltpu.SemaphoreType.DMA,
                pltpu.SemaphoreType.DMA,
                pltpu.SemaphoreType.DMA,
            ],
        )
    return _EDGE_PASS_CACHE["k"]


def kernel(x, edge_index, edge_attr, W, b, U0, b0, U1, b1, U2, b2):
    src3d = edge_index[0].reshape(NS, NCHUNK, CHUNK)
    dst3d = edge_index[1].reshape(NS, NCHUNK, CHUNK)

    h, ha, hb = _input_projection(x, W, b, U1, b1)
    e2a, e2b = _edge_projection(edge_attr, U2, b2)

    def depth_body(_, carry):
        h, ha, hb = carry
        agga, aggb = _edge_pass_kernel()(ha, hb, e2a, e2b, src3d, dst3d)
        return tuple(_node_update(h, agga, aggb, U0, b0, U1, b1))

    h, ha, hb = lax.fori_loop(0, DEPTH, depth_body, (h, ha, hb))
    return h
